# baseline (speedup 1.0000x reference)
import jax
import jax.numpy as jnp
from jax import lax
from jax.experimental import pallas as pl

N = 35
E = 1190
NV = 6
C1_IN, C1_OUT = 1, 36
C2_IN, C2_OUT = 36, 24
C3_IN, C3_OUT = 24, 5

F32 = jnp.float32
BF16 = jnp.bfloat16


def _expand_mat(in_c, out_c):
    i = lax.broadcasted_iota(jnp.int32, (in_c, in_c * out_c), 0)
    j = lax.broadcasted_iota(jnp.int32, (in_c, in_c * out_c), 1)
    return (j // out_c == i).astype(BF16)


def _fold_mat(in_c, out_c):
    j = lax.broadcasted_iota(jnp.int32, (in_c * out_c, out_c), 0)
    o = lax.broadcasted_iota(jnp.int32, (in_c * out_c, out_c), 1)
    return (j % out_c == o).astype(BF16)


def _dot(a, b):
    return jnp.dot(a, b, preferred_element_type=F32)


def _dot_t(a, b):
    return lax.dot_general(a, b, (((0,), (0,)), ((), ())),
                           preferred_element_type=F32)


def _split3(b):
    b1 = b.astype(BF16)
    r = b - b1.astype(F32)
    b2 = r.astype(BF16)
    b3 = (r - b2.astype(F32)).astype(BF16)
    return b1, b2, b3


def _dot_exact(a_bf, b):
    b1, b2, b3 = _split3(b)
    return (_dot(a_bf, b1) + _dot(a_bf, b2)) + _dot(a_bf, b3)


def _dot_t_exact(a_bf, b):
    b1, b2, b3 = _split3(b)
    return (_dot_t(a_bf, b1) + _dot_t(a_bf, b2)) + _dot_t(a_bf, b3)


def _dot_hilo(a_bf, p):
    p_hi = p.astype(BF16)
    p_lo = (p - p_hi.astype(F32)).astype(BF16)
    return _dot(a_bf, p_hi) + _dot(a_bf, p_lo)


def _dot_exact_l(a, b_bf):
    a1, a2, a3 = _split3(a)
    return (_dot(a1, b_bf) + _dot(a2, b_bf)) + _dot(a3, b_bf)


def _fused_kernel(x_ref, ea_ref, ei_ref,
                  w1_ref, b1_ref, r1_ref, c1_ref,
                  w2_ref, b2_ref, r2_ref, c2_ref,
                  w3_ref, b3_ref, r3_ref, c3_ref,
                  out_ref):
    def trunc(a):
        return a.astype(BF16).astype(F32)

    def row(ref):
        return ref[...].reshape(1, -1)

    src = ei_ref[0:1, :]
    dst = ei_ref[1:2, :]
    iota_ne = lax.broadcasted_iota(jnp.int32, (N, E), 0)
    D = (dst == iota_ne).astype(BF16)
    G = (src == iota_ne).astype(BF16)
    cnt = _dot(D, jnp.ones((E, 1), BF16))
    inv_cnt = 1.0 / jnp.maximum(cnt, 1.0)

    ea_bf = ea_ref[...].astype(BF16)
    x = x_ref[...]

    h1 = jnp.maximum(_dot(ea_bf, w1_ref[...].astype(BF16)) + row(b1_ref), 0.0)
    xg1 = _dot_t_exact(G, x)
    msg1 = xg1 * h1
    agg1 = _dot_exact(D, msg1) * inv_cnt
    n1 = jnp.maximum(x * r1_ref[...] + agg1 + row(c1_ref), 0.0)

    h2 = jnp.maximum(_dot(ea_bf, w2_ref[...].astype(BF16)) + row(b2_ref), 0.0)
    xg2 = _dot_t(G, n1.astype(BF16))
    xe2 = _dot(xg2.astype(BF16), _expand_mat(C2_IN, C2_OUT))
    sag2 = _dot_hilo(D, trunc(h2) * xe2)
    agg2 = _dot_exact_l(sag2, _fold_mat(C2_IN, C2_OUT)) * inv_cnt
    n2 = jnp.maximum(_dot(n1.astype(BF16), r2_ref[...].astype(BF16))
                     + agg2 + row(c2_ref), 0.0)

    h3 = jnp.maximum(_dot(ea_bf, w3_ref[...].astype(BF16)) + row(b3_ref), 0.0)
    xg3 = _dot_t(G, n2.astype(BF16))
    xe3 = _dot(xg3.astype(BF16), _expand_mat(C3_IN, C3_OUT))
    sag3 = _dot_hilo(D, trunc(h3) * xe3)
    agg3 = _dot_exact_l(sag3, _fold_mat(C3_IN, C3_OUT)) * inv_cnt
    n3 = jnp.maximum(_dot(n2.astype(BF16), r3_ref[...].astype(BF16))
                     + agg3 + row(c3_ref), 0.0)

    diff = jnp.abs(n3[:, None, :] - n3[None, :, :])
    out_ref[...] = jnp.sum(diff, axis=2)


def kernel(x, edge_attr, edge_index,
           lin1_W, lin1_b, conv1_root, conv1_bias,
           lin2_W, lin2_b, conv2_root, conv2_bias,
           lin3_W, lin3_b, conv3_root, conv3_bias):
    args = (
        x, edge_attr, edge_index.astype(jnp.int32),
        lin1_W, lin1_b, conv1_root, conv1_bias,
        lin2_W, lin2_b, conv2_root, conv2_bias,
        lin3_W, lin3_b, conv3_root, conv3_bias,
    )
    return pl.pallas_call(
        _fused_kernel,
        out_shape=jax.ShapeDtypeStruct((N, N), F32),
    )(*args)
